# baseline (speedup 1.0000x reference)
import jax
import jax.numpy as jnp
from jax.experimental import pallas as pl

MAXLEN = 3
EMBED_DIM = 640
ROW = MAXLEN * EMBED_DIM


def _add_kernel(x_ref, pos_ref, o_ref):
    o_ref[...] = x_ref[...] + pos_ref[...]


def kernel(x, pos_table):
    n = x.shape[0]
    x2 = x.reshape(n, ROW)
    pos2 = pos_table.reshape(1, ROW)
    blk = 1024
    out = pl.pallas_call(
        _add_kernel,
        grid=(n // blk,),
        in_specs=[
            pl.BlockSpec((blk, ROW), lambda i: (i, 0)),
            pl.BlockSpec((1, ROW), lambda i: (0, 0)),
        ],
        out_specs=pl.BlockSpec((blk, ROW), lambda i: (i, 0)),
        out_shape=jax.ShapeDtypeStruct((n, ROW), x.dtype),
    )(x2, pos2)
    return out.reshape(n, MAXLEN, EMBED_DIM)


# speedup vs baseline: 1.3908x; 1.3908x over previous
import jax
import jax.numpy as jnp
from jax.experimental import pallas as pl

MAXLEN = 3
EMBED_DIM = 640
ROW = MAXLEN * EMBED_DIM


def _add_kernel(x_ref, pos_ref, o_ref):
    o_ref[...] = x_ref[...] + pos_ref[...]


def kernel(x, pos_table):
    n = x.shape[0]
    blk = 1024
    out = pl.pallas_call(
        _add_kernel,
        grid=(n // blk,),
        in_specs=[
            pl.BlockSpec((blk, MAXLEN, EMBED_DIM), lambda i: (i, 0, 0)),
            pl.BlockSpec((MAXLEN, EMBED_DIM), lambda i: (0, 0)),
        ],
        out_specs=pl.BlockSpec((blk, MAXLEN, EMBED_DIM), lambda i: (i, 0, 0)),
        out_shape=jax.ShapeDtypeStruct(x.shape, x.dtype),
    )(x, pos_table)
    return out
